# baseline (speedup 1.0000x reference)
import functools

import jax
import jax.numpy as jnp
from jax import lax
from jax.experimental import pallas as pl
from jax.experimental.pallas import tpu as pltpu
from jax.experimental.pallas import tpu_sc as plsc

NUM_IN = 1024
NUM_OUT = 1024
NUM_MID = 4096
FAN_IN = 16
BATCH = 1024

TJ = 512
NC, NS, L = 2, 16, 16
NW = NC * NS
JW = NUM_MID // NW
CUNROLL = 4


def _mm_in_body(x_ref, w_ref, b_ref, o_ref):
    acc = lax.dot_general(
        w_ref[...], x_ref[...], (((1,), (1,)), ((), ())),
        preferred_element_type=jnp.float32)
    o_ref[...] = jnp.maximum(acc + b_ref[...].reshape(-1, 1), 0.0)


def _mm_out_body(a_ref, w_ref, b_ref, o_ref):
    acc = lax.dot_general(
        a_ref[...], w_ref[...], (((0,), (1,)), ((), ())),
        preferred_element_type=jnp.float32)
    o_ref[...] = acc + b_ref[...].reshape(1, -1)


def _mm_in(x, W_in, b_in):
    return pl.pallas_call(
        _mm_in_body,
        grid=(NUM_MID // TJ,),
        in_specs=[
            pl.BlockSpec((BATCH, NUM_IN), lambda i: (0, 0)),
            pl.BlockSpec((TJ, NUM_IN), lambda i: (i, 0)),
            pl.BlockSpec((TJ, 1), lambda i: (i, 0)),
        ],
        out_specs=pl.BlockSpec((TJ, BATCH), lambda i: (i, 0)),
        out_shape=jax.ShapeDtypeStruct((NUM_MID, BATCH), jnp.float32),
    )(x, W_in, b_in.reshape(NUM_MID, 1))


def _mm_out(a, W_out, b_out):
    return pl.pallas_call(
        _mm_out_body,
        grid=(NUM_OUT // TJ,),
        in_specs=[
            pl.BlockSpec((NUM_MID, BATCH), lambda i: (0, 0)),
            pl.BlockSpec((TJ, NUM_MID), lambda i: (i, 0)),
            pl.BlockSpec((1, TJ), lambda i: (0, i)),
        ],
        out_specs=pl.BlockSpec((BATCH, TJ), lambda i: (0, i)),
        out_shape=jax.ShapeDtypeStruct((BATCH, NUM_OUT), jnp.float32),
    )(a, W_out, b_out.reshape(1, NUM_OUT))


def _cond_sc_body(table, w_hbm, b_hbm, idx_hbm, out_hbm,
                  idx_v, w_v, b_v, rows_v, out_v,
                  gsem0, gsem1, osem0, osem1):
    gsems = (gsem0, gsem1)
    osems = (osem0, osem1)
    wid = lax.axis_index("s") * NC + lax.axis_index("c")
    base = wid * JW

    pltpu.sync_copy(idx_hbm.at[pl.ds(base, JW)], idx_v)
    pltpu.sync_copy(w_hbm.at[pl.ds(base, JW)], w_v)
    pltpu.sync_copy(b_hbm.at[pl.ds(base, JW)], b_v)

    def start_gather(j, p):
        pltpu.make_async_copy(
            table.at[idx_v.at[j]], rows_v.at[p], gsems[p]).start()

    def wait_gather(p):
        pltpu.make_async_copy(
            table.at[idx_v.at[0]], rows_v.at[p], gsems[p]).wait()

    def compute(j, p):
        jv = jnp.full((L,), j, jnp.int32)
        bj = plsc.load_gather(b_v, [jv])
        wks = [plsc.load_gather(w_v, [jv, jnp.full((L,), k, jnp.int32)])
               for k in range(FAN_IN)]

        def cbody(ci, carry):
            for u in range(CUNROLL):
                off = (ci * CUNROLL + u) * L
                acc = bj
                for k in range(FAN_IN):
                    acc = acc + wks[k] * rows_v[p, k, pl.ds(off, L)]
                out_v[p, pl.ds(off, L)] = jnp.maximum(acc, 0.0)
            return carry

        lax.fori_loop(0, BATCH // (L * CUNROLL), cbody, 0)

    start_gather(0, 0)
    start_gather(1, 1)

    def jbody(j2, carry):
        for p in range(2):
            j = 2 * j2 + p
            wait_gather(p)

            @pl.when(j2 > 0)
            def _():
                pltpu.make_async_copy(
                    out_v.at[p], out_hbm.at[base + j], osems[p]).wait()

            compute(j, p)
            pltpu.make_async_copy(
                out_v.at[p], out_hbm.at[base + j], osems[p]).start()

            @pl.when(j + 2 < JW)
            def _():
                start_gather(j + 2, p)
        return carry

    lax.fori_loop(0, JW // 2, jbody, 0)

    for p in range(2):
        pltpu.make_async_copy(
            out_v.at[p], out_hbm.at[base + JW - 2 + p], osems[p]).wait()


_cond_sc = pl.kernel(
    _cond_sc_body,
    out_type=jax.ShapeDtypeStruct((NUM_MID, BATCH), jnp.float32),
    mesh=plsc.VectorSubcoreMesh(core_axis_name="c", subcore_axis_name="s"),
    compiler_params=pltpu.CompilerParams(needs_layout_passes=False),
    scratch_types=[
        pltpu.VMEM((JW, FAN_IN), jnp.int32),
        pltpu.VMEM((JW, FAN_IN), jnp.float32),
        pltpu.VMEM((JW,), jnp.float32),
        pltpu.VMEM((2, FAN_IN, BATCH), jnp.float32),
        pltpu.VMEM((2, BATCH), jnp.float32),
        pltpu.SemaphoreType.DMA,
        pltpu.SemaphoreType.DMA,
        pltpu.SemaphoreType.DMA,
        pltpu.SemaphoreType.DMA,
    ],
)


@jax.jit
def kernel(x, W_in, b_in, W_mid0, b_mid0, W_mid1, b_mid1, W_out, b_out,
           indx_seqs):
    a = _mm_in(x, W_in, b_in)
    a = _cond_sc(a, W_mid0, b_mid0, indx_seqs)
    a = _cond_sc(a, W_mid1, b_mid1, indx_seqs)
    return _mm_out(a, W_out, b_out)


# speedup vs baseline: 2.4558x; 2.4558x over previous
import functools

import jax
import jax.numpy as jnp
from jax import lax
from jax.experimental import pallas as pl
from jax.experimental.pallas import tpu as pltpu
from jax.experimental.pallas import tpu_sc as plsc

NUM_IN = 1024
NUM_OUT = 1024
NUM_MID = 4096
FAN_IN = 16
BATCH = 1024

TJ = 512
NC, NS, L = 2, 16, 16
NW = NC * NS
JW = NUM_MID // NW


def _sbuild_body(w0_hbm, w1_hbm, idx_hbm, s0_hbm, s1_hbm,
                 idx_v, w_v, buf0, buf1, sem0, sem1):
    sems = (sem0, sem1)
    bufs = (buf0, buf1)
    wid = lax.axis_index("s") * NC + lax.axis_index("c")
    base = wid * JW

    pltpu.sync_copy(idx_hbm.at[pl.ds(base, JW)], idx_v)

    zero = jnp.zeros((L,), jnp.float32)

    def zbody(i, c):
        buf0[pl.ds(i * L, L)] = zero
        buf1[pl.ds(i * L, L)] = zero
        return c

    lax.fori_loop(0, NUM_MID // L, zbody, 0)

    def build(w_hbm, s_hbm):
        pltpu.sync_copy(w_hbm.at[pl.ds(base, JW)], w_v)

        def rbody(r2, c):
            for p in range(2):
                r = 2 * r2 + p

                @pl.when(r2 > 0)
                def _():
                    pltpu.make_async_copy(
                        bufs[p], s_hbm.at[0], sems[p]).wait()
                    plsc.store_scatter(bufs[p], [idx_v[r - 2, :]], zero)

                plsc.addupdate_scatter(
                    bufs[p], [idx_v[r, :]], w_v[r, :])
                pltpu.make_async_copy(
                    bufs[p], s_hbm.at[base + r], sems[p]).start()
            return c

        lax.fori_loop(0, JW // 2, rbody, 0)

        for p in range(2):
            pltpu.make_async_copy(bufs[p], s_hbm.at[0], sems[p]).wait()
            plsc.store_scatter(bufs[p], [idx_v[JW - 2 + p, :]], zero)

    build(w0_hbm, s0_hbm)
    build(w1_hbm, s1_hbm)


_sbuild = pl.kernel(
    _sbuild_body,
    out_type=(jax.ShapeDtypeStruct((NUM_MID, NUM_MID), jnp.float32),
              jax.ShapeDtypeStruct((NUM_MID, NUM_MID), jnp.float32)),
    mesh=plsc.VectorSubcoreMesh(core_axis_name="c", subcore_axis_name="s"),
    compiler_params=pltpu.CompilerParams(needs_layout_passes=False),
    scratch_types=[
        pltpu.VMEM((JW, FAN_IN), jnp.int32),
        pltpu.VMEM((JW, FAN_IN), jnp.float32),
        pltpu.VMEM((NUM_MID,), jnp.float32),
        pltpu.VMEM((NUM_MID,), jnp.float32),
        pltpu.SemaphoreType.DMA,
        pltpu.SemaphoreType.DMA,
    ],
)


def _mm_in_body(x_ref, w_ref, b_ref, o_ref):
    acc = lax.dot_general(
        w_ref[...], x_ref[...], (((1,), (1,)), ((), ())),
        preferred_element_type=jnp.float32)
    o_ref[...] = jnp.maximum(
        acc + b_ref[...].reshape(-1, 1), 0.0).astype(jnp.bfloat16)


def _cond_mm_body(s_ref, a_ref, b_ref, o_ref):
    s16 = s_ref[...].astype(jnp.bfloat16)
    acc = lax.dot_general(
        s16, a_ref[...], (((1,), (0,)), ((), ())),
        preferred_element_type=jnp.float32)
    o_ref[...] = jnp.maximum(
        acc + b_ref[...].reshape(-1, 1), 0.0).astype(jnp.bfloat16)


def _mm_out_body(a_ref, w_ref, b_ref, o_ref):
    w16 = w_ref[...].astype(jnp.bfloat16)
    acc = lax.dot_general(
        a_ref[...], w16, (((0,), (1,)), ((), ())),
        preferred_element_type=jnp.float32)
    o_ref[...] = acc + b_ref[...].reshape(1, -1)


def _mm_in(x, W_in, b_in):
    return pl.pallas_call(
        _mm_in_body,
        grid=(NUM_MID // TJ,),
        in_specs=[
            pl.BlockSpec((BATCH, NUM_IN), lambda i: (0, 0)),
            pl.BlockSpec((TJ, NUM_IN), lambda i: (i, 0)),
            pl.BlockSpec((TJ, 1), lambda i: (i, 0)),
        ],
        out_specs=pl.BlockSpec((TJ, BATCH), lambda i: (i, 0)),
        out_shape=jax.ShapeDtypeStruct((NUM_MID, BATCH), jnp.bfloat16),
    )(x, W_in, b_in.reshape(NUM_MID, 1))


def _cond_mm(s, a, b):
    return pl.pallas_call(
        _cond_mm_body,
        grid=(NUM_MID // TJ,),
        in_specs=[
            pl.BlockSpec((TJ, NUM_MID), lambda i: (i, 0)),
            pl.BlockSpec((NUM_MID, BATCH), lambda i: (0, 0)),
            pl.BlockSpec((TJ, 1), lambda i: (i, 0)),
        ],
        out_specs=pl.BlockSpec((TJ, BATCH), lambda i: (i, 0)),
        out_shape=jax.ShapeDtypeStruct((NUM_MID, BATCH), jnp.bfloat16),
    )(s, a, b.reshape(NUM_MID, 1))


def _mm_out(a, W_out, b_out):
    return pl.pallas_call(
        _mm_out_body,
        grid=(NUM_OUT // TJ,),
        in_specs=[
            pl.BlockSpec((NUM_MID, BATCH), lambda i: (0, 0)),
            pl.BlockSpec((TJ, NUM_MID), lambda i: (i, 0)),
            pl.BlockSpec((1, TJ), lambda i: (0, i)),
        ],
        out_specs=pl.BlockSpec((BATCH, TJ), lambda i: (0, i)),
        out_shape=jax.ShapeDtypeStruct((BATCH, NUM_OUT), jnp.float32),
    )(a, W_out, b_out.reshape(1, NUM_OUT))


@jax.jit
def kernel(x, W_in, b_in, W_mid0, b_mid0, W_mid1, b_mid1, W_out, b_out,
           indx_seqs):
    s0, s1 = _sbuild(W_mid0, W_mid1, indx_seqs)
    a = _mm_in(x, W_in, b_in)
    a = _cond_mm(s0, a, b_mid0)
    a = _cond_mm(s1, a, b_mid1)
    return _mm_out(a, W_out, b_out)


# speedup vs baseline: 2.8392x; 1.1561x over previous
import functools

import jax
import jax.numpy as jnp
from jax import lax
from jax.experimental import pallas as pl
from jax.experimental.pallas import tpu as pltpu
from jax.experimental.pallas import tpu_sc as plsc

NUM_IN = 1024
NUM_OUT = 1024
NUM_MID = 4096
FAN_IN = 16
BATCH = 1024

TJ = 512
NC, NS, L = 2, 16, 16
NW = NC * NS
JW = NUM_MID // NW


def _sbuild_body(wt_hbm, idxt_hbm, s_hbm, idx_v, w_v, buf0, buf1,
                 sem0, sem1):
    sems = (sem0, sem1)
    bufs = (buf0, buf1)
    wid = lax.axis_index("s") * NC + lax.axis_index("c")
    base = wid * JW

    pltpu.sync_copy(idxt_hbm.at[:, pl.ds(base, JW)], idx_v)
    pltpu.sync_copy(wt_hbm.at[:, pl.ds(base, JW)], w_v)

    zero = jnp.zeros((L,), jnp.float32)
    lane = lax.iota(jnp.int32, L)

    def zbody(i, c):
        buf0[pl.ds(i * L, L)] = zero
        buf1[pl.ds(i * L, L)] = zero
        return c

    lax.fori_loop(0, NUM_MID // L, zbody, 0)

    def row_idx(r):
        return plsc.load_gather(idx_v, [lane, jnp.full((L,), r, jnp.int32)])

    def rbody(r2, c):
        for p in range(2):
            r = 2 * r2 + p

            @pl.when(r2 > 0)
            def _():
                pltpu.make_async_copy(bufs[p], s_hbm.at[0], sems[p]).wait()
                plsc.store_scatter(bufs[p], [row_idx(r - 2)], zero)

            wr = plsc.load_gather(w_v, [lane, jnp.full((L,), r, jnp.int32)])
            plsc.addupdate_scatter(bufs[p], [row_idx(r)], wr)
            pltpu.make_async_copy(
                bufs[p], s_hbm.at[base + r], sems[p]).start()
        return c

    lax.fori_loop(0, JW // 2, rbody, 0)

    for p in range(2):
        pltpu.make_async_copy(bufs[p], s_hbm.at[0], sems[p]).wait()


_sbuild = pl.kernel(
    _sbuild_body,
    out_type=jax.ShapeDtypeStruct((NUM_MID, NUM_MID), jnp.float32),
    mesh=plsc.VectorSubcoreMesh(core_axis_name="c", subcore_axis_name="s"),
    compiler_params=pltpu.CompilerParams(needs_layout_passes=False),
    scratch_types=[
        pltpu.VMEM((FAN_IN, JW), jnp.int32),
        pltpu.VMEM((FAN_IN, JW), jnp.float32),
        pltpu.VMEM((NUM_MID,), jnp.float32),
        pltpu.VMEM((NUM_MID,), jnp.float32),
        pltpu.SemaphoreType.DMA,
        pltpu.SemaphoreType.DMA,
    ],
)


def _mm_in_body(x_ref, w_ref, b_ref, o_ref):
    acc = lax.dot_general(
        w_ref[...], x_ref[...], (((1,), (1,)), ((), ())),
        preferred_element_type=jnp.float32)
    o_ref[...] = jnp.maximum(
        acc + b_ref[...].reshape(-1, 1), 0.0).astype(jnp.bfloat16)


def _cond_mm_body(s_ref, a_ref, b_ref, o_ref):
    s16 = s_ref[...].astype(jnp.bfloat16)
    acc = lax.dot_general(
        s16, a_ref[...], (((1,), (0,)), ((), ())),
        preferred_element_type=jnp.float32)
    o_ref[...] = jnp.maximum(
        acc + b_ref[...].reshape(-1, 1), 0.0).astype(jnp.bfloat16)


def _mm_out_body(a_ref, w_ref, b_ref, o_ref):
    w16 = w_ref[...].astype(jnp.bfloat16)
    acc = lax.dot_general(
        a_ref[...], w16, (((0,), (1,)), ((), ())),
        preferred_element_type=jnp.float32)
    o_ref[...] = acc + b_ref[...].reshape(1, -1)


def _mm_in(x, W_in, b_in):
    return pl.pallas_call(
        _mm_in_body,
        grid=(NUM_MID // TJ,),
        in_specs=[
            pl.BlockSpec((BATCH, NUM_IN), lambda i: (0, 0)),
            pl.BlockSpec((TJ, NUM_IN), lambda i: (i, 0)),
            pl.BlockSpec((1, TJ), lambda i: (0, i)),
        ],
        out_specs=pl.BlockSpec((TJ, BATCH), lambda i: (i, 0)),
        out_shape=jax.ShapeDtypeStruct((NUM_MID, BATCH), jnp.bfloat16),
    )(x, W_in, b_in.reshape(1, NUM_MID))


def _cond_mm(s, a, b):
    return pl.pallas_call(
        _cond_mm_body,
        grid=(NUM_MID // TJ,),
        in_specs=[
            pl.BlockSpec((TJ, NUM_MID), lambda i: (i, 0)),
            pl.BlockSpec((NUM_MID, BATCH), lambda i: (0, 0)),
            pl.BlockSpec((1, TJ), lambda i: (0, i)),
        ],
        out_specs=pl.BlockSpec((TJ, BATCH), lambda i: (i, 0)),
        out_shape=jax.ShapeDtypeStruct((NUM_MID, BATCH), jnp.bfloat16),
    )(s, a, b.reshape(1, NUM_MID))


def _mm_out(a, W_out, b_out):
    return pl.pallas_call(
        _mm_out_body,
        grid=(NUM_OUT // TJ,),
        in_specs=[
            pl.BlockSpec((NUM_MID, BATCH), lambda i: (0, 0)),
            pl.BlockSpec((TJ, NUM_MID), lambda i: (i, 0)),
            pl.BlockSpec((1, TJ), lambda i: (0, i)),
        ],
        out_specs=pl.BlockSpec((BATCH, TJ), lambda i: (0, i)),
        out_shape=jax.ShapeDtypeStruct((BATCH, NUM_OUT), jnp.float32),
    )(a, W_out, b_out.reshape(1, NUM_OUT))


@jax.jit
def kernel(x, W_in, b_in, W_mid0, b_mid0, W_mid1, b_mid1, W_out, b_out,
           indx_seqs):
    idx_t = indx_seqs.T
    s0 = _sbuild(W_mid0.T, idx_t)
    s1 = _sbuild(W_mid1.T, idx_t)
    a = _mm_in(x, W_in, b_in)
    a = _cond_mm(s0, a, b_mid0)
    a = _cond_mm(s1, a, b_mid1)
    return _mm_out(a, W_out, b_out)


# speedup vs baseline: 3.2947x; 1.1604x over previous
import functools

import jax
import jax.numpy as jnp
from jax import lax
from jax.experimental import pallas as pl
from jax.experimental.pallas import tpu as pltpu
from jax.experimental.pallas import tpu_sc as plsc

NUM_IN = 1024
NUM_OUT = 1024
NUM_MID = 4096
FAN_IN = 16
BATCH = 1024

TJ = 512
NC, NS, L = 2, 16, 16
NW = NC * NS
JW = NUM_MID // NW


JWP = JW // 2


def _sbuild_body(wt_hbm, idxt_hbm, s_hbm, idx_v, w_v, fmerge, buf0, buf1,
                 sem0, sem1):
    sems = (sem0, sem1)
    bufs = (buf0, buf1)
    wid = lax.axis_index("s") * NC + lax.axis_index("c")
    base = wid * JW

    pltpu.sync_copy(idxt_hbm.at[:, pl.ds(base, JW)], idx_v)
    pltpu.sync_copy(wt_hbm.at[:, pl.ds(base, JW)], w_v)

    zero = jnp.zeros((L,), jnp.float32)
    izero = jnp.zeros((L,), jnp.int32)
    lane = lax.iota(jnp.int32, L)

    def zbody(i, c):
        fmerge[pl.ds(i * L, L)] = zero
        buf0[pl.ds(i * L, L)] = izero
        buf1[pl.ds(i * L, L)] = izero
        return c

    lax.fori_loop(0, NUM_MID // L, zbody, 0)

    def col_of(col, ref):
        return plsc.load_gather(ref, [lane, jnp.full((L,), col, jnp.int32)])

    def half_bits(col, h):
        idxv = col_of(col, idx_v)
        wv = col_of(col, w_v)
        plsc.addupdate_scatter(fmerge, [idxv], wv)
        wsum = plsc.load_gather(fmerge, [idxv])
        plsc.store_scatter(fmerge, [idxv], zero)
        dupc = izero
        for k2 in range(FAN_IN):
            ib = plsc.load_gather(
                idx_v, [jnp.full((L,), k2, jnp.int32),
                        jnp.full((L,), col, jnp.int32)])
            dupc = dupc + jnp.where(
                (idxv == ib) & (lane > k2), 1, 0).astype(jnp.int32)
        bits = plsc.bitcast(wsum, jnp.uint32)
        bits = (bits + 0x7FFF + ((bits >> 16) & 1)) >> 16
        if h:
            bits = bits << 16
        bits = jnp.where(dupc > 0, jnp.zeros((L,), jnp.uint32), bits)
        return idxv, plsc.bitcast(bits, jnp.int32)

    def rbody(r2, c):
        for p in range(2):
            r = 2 * r2 + p

            @pl.when(r2 > 0)
            def _():
                pltpu.make_async_copy(bufs[p], s_hbm.at[0], sems[p]).wait()
                for h in range(2):
                    oidx = col_of(2 * (r - 2) + h, idx_v)
                    plsc.store_scatter(bufs[p], [oidx], izero)

            for h in range(2):
                idxv, bits = half_bits(2 * r + h, h)
                plsc.addupdate_scatter(bufs[p], [idxv], bits)
            pltpu.make_async_copy(
                bufs[p], s_hbm.at[wid * JWP + r], sems[p]).start()
        return c

    lax.fori_loop(0, JWP // 2, rbody, 0)

    for p in range(2):
        pltpu.make_async_copy(bufs[p], s_hbm.at[0], sems[p]).wait()


_sbuild = pl.kernel(
    _sbuild_body,
    out_type=jax.ShapeDtypeStruct((NUM_MID // 2, NUM_MID), jnp.int32),
    mesh=plsc.VectorSubcoreMesh(core_axis_name="c", subcore_axis_name="s"),
    compiler_params=pltpu.CompilerParams(needs_layout_passes=False),
    scratch_types=[
        pltpu.VMEM((FAN_IN, JW), jnp.int32),
        pltpu.VMEM((FAN_IN, JW), jnp.float32),
        pltpu.VMEM((NUM_MID,), jnp.float32),
        pltpu.VMEM((NUM_MID,), jnp.int32),
        pltpu.VMEM((NUM_MID,), jnp.int32),
        pltpu.SemaphoreType.DMA,
        pltpu.SemaphoreType.DMA,
    ],
)


def _mm_in_body(x_ref, w_ref, b_ref, o_ref):
    acc = lax.dot_general(
        w_ref[...], x_ref[...], (((1,), (1,)), ((), ())),
        preferred_element_type=jnp.float32)
    o_ref[...] = jnp.maximum(
        acc + b_ref[...].reshape(-1, 1), 0.0).astype(jnp.bfloat16)


def _cond_mm_body(s_ref, a_ref, b_ref, o_ref):
    s16 = pltpu.bitcast(s_ref[...], jnp.bfloat16)
    acc = lax.dot_general(
        s16, a_ref[...], (((1,), (0,)), ((), ())),
        preferred_element_type=jnp.float32)
    o_ref[...] = jnp.maximum(
        acc + b_ref[...].reshape(-1, 1), 0.0).astype(jnp.bfloat16)


def _mm_out_body(a_ref, w_ref, b_ref, o_ref):
    w16 = w_ref[...].astype(jnp.bfloat16)
    acc = lax.dot_general(
        a_ref[...], w16, (((0,), (1,)), ((), ())),
        preferred_element_type=jnp.float32)
    o_ref[...] = acc + b_ref[...].reshape(1, -1)


def _mm_in(x, W_in, b_in):
    return pl.pallas_call(
        _mm_in_body,
        grid=(NUM_MID // TJ,),
        in_specs=[
            pl.BlockSpec((BATCH, NUM_IN), lambda i: (0, 0)),
            pl.BlockSpec((TJ, NUM_IN), lambda i: (i, 0)),
            pl.BlockSpec((1, TJ), lambda i: (0, i)),
        ],
        out_specs=pl.BlockSpec((TJ, BATCH), lambda i: (i, 0)),
        out_shape=jax.ShapeDtypeStruct((NUM_MID, BATCH), jnp.bfloat16),
    )(x, W_in, b_in.reshape(1, NUM_MID))


def _cond_mm(s, a, b):
    return pl.pallas_call(
        _cond_mm_body,
        grid=(NUM_MID // TJ,),
        in_specs=[
            pl.BlockSpec((TJ // 2, NUM_MID), lambda i: (i, 0)),
            pl.BlockSpec((NUM_MID, BATCH), lambda i: (0, 0)),
            pl.BlockSpec((1, TJ), lambda i: (0, i)),
        ],
        out_specs=pl.BlockSpec((TJ, BATCH), lambda i: (i, 0)),
        out_shape=jax.ShapeDtypeStruct((NUM_MID, BATCH), jnp.bfloat16),
    )(s, a, b.reshape(1, NUM_MID))


def _mm_out(a, W_out, b_out):
    return pl.pallas_call(
        _mm_out_body,
        grid=(NUM_OUT // TJ,),
        in_specs=[
            pl.BlockSpec((NUM_MID, BATCH), lambda i: (0, 0)),
            pl.BlockSpec((TJ, NUM_MID), lambda i: (i, 0)),
            pl.BlockSpec((1, TJ), lambda i: (0, i)),
        ],
        out_specs=pl.BlockSpec((BATCH, TJ), lambda i: (0, i)),
        out_shape=jax.ShapeDtypeStruct((BATCH, NUM_OUT), jnp.float32),
    )(a, W_out, b_out.reshape(1, NUM_OUT))


@jax.jit
def kernel(x, W_in, b_in, W_mid0, b_mid0, W_mid1, b_mid1, W_out, b_out,
           indx_seqs):
    idx_t = indx_seqs.T
    s0 = _sbuild(W_mid0.T, idx_t)
    s1 = _sbuild(W_mid1.T, idx_t)
    a = _mm_in(x, W_in, b_in)
    a = _cond_mm(s0, a, b_mid0)
    a = _cond_mm(s1, a, b_mid1)
    return _mm_out(a, W_out, b_out)
